# baseline (speedup 1.0000x reference)
import jax
import jax.numpy as jnp
from jax.experimental import pallas as pl


def _conv(x, edge_index, edge_weight, W, b):
    n = x.shape[0]
    src = edge_index[0]
    dst = edge_index[1]
    loop = jnp.arange(n, dtype=src.dtype)
    src = jnp.concatenate([src, loop])
    dst = jnp.concatenate([dst, loop])
    ew = jnp.concatenate([edge_weight, jnp.ones((n,), dtype=edge_weight.dtype)])
    deg = jax.ops.segment_sum(ew, dst, num_segments=n)
    safe_deg = jnp.where(deg > 0, deg, 1.0)
    dinv = jnp.where(deg > 0, 1.0 / jnp.sqrt(safe_deg), 0.0)
    norm = dinv[src] * ew * dinv[dst]
    h = x @ W
    msg = h[src] * norm[:, None]
    out = jax.ops.segment_sum(msg, dst, num_segments=n) + b
    return out


def kernel(x, edge_index, edge_weight, W1, b1, W2, b2):
    h = jax.nn.relu(_conv(x, edge_index, edge_weight, W1, b1))
    out = _conv(h, edge_index, edge_weight, W2, b2)
    return (h, out)


# speedup vs baseline: 16.2983x; 16.2983x over previous
import functools

import jax
import jax.numpy as jnp
from jax import lax
from jax.experimental import pallas as pl
from jax.experimental.pallas import tpu as pltpu
from jax.experimental.pallas import tpu_sc as plsc

N = 10000
D = 256
H1 = 32
G = 16
NC = 2
NS = 16
NW = NC * NS
CH = 128
NCH = 40
EPW = NCH * CH
EP = NW * EPW
NP = 10240
RPT = NP // NS

@functools.lru_cache(maxsize=None)
def _sc_mesh():
    return plsc.VectorSubcoreMesh(
        core_axis_name="c", subcore_axis_name="s", num_cores=NC, num_subcores=NS
    )


def _deg_body(dstp_h, ewp_h, z16_h, degp_h, dst_v, ew_v, msg_v, acc):
    cid = lax.axis_index("c")
    sid = lax.axis_index("s")
    wid = sid * NC + cid
    row0 = pl.ds(sid * RPT, RPT)
    pltpu.sync_copy(z16_h.at[row0], acc.at[row0])
    pltpu.sync_copy(dstp_h.at[wid], dst_v)
    pltpu.sync_copy(ewp_h.at[wid], ew_v)
    plsc.subcore_barrier()

    @pl.loop(0, NCH)
    def _chunk(j):
        for g in range(CH // 16):
            ewg = ew_v[j, pl.ds(g * 16, 16)]
            for e2 in range(16):
                w = ewg[e2]
                msg_v[g * 16 + e2, :] = lax.broadcast_in_dim(w, (16,), ())
        pltpu.sync_copy(msg_v, acc.at[dst_v.at[j]], add=True)

    plsc.subcore_barrier()
    pltpu.sync_copy(acc.at[row0], degp_h.at[cid, row0])


@functools.lru_cache(maxsize=None)
def _deg_call():
    return pl.kernel(
        _deg_body,
        out_type=jax.ShapeDtypeStruct((NC, NP, G), jnp.float32),
        mesh=_sc_mesh(),
        compiler_params=pltpu.CompilerParams(use_tc_tiling_on_sc=False),
        scratch_types=[
            pltpu.VMEM((NCH, CH), jnp.int32),
            pltpu.VMEM((NCH, CH), jnp.float32),
            pltpu.VMEM((CH, G), jnp.float32),
            pltpu.VMEM_SHARED((NP, G), jnp.float32),
        ],
    )


def _edge_body(width, srcp_h, dstp_h, ewp_h, feat_h, z_h, outp_h,
               src_v, dst_v, ew_v, rows_v, acc):
    cid = lax.axis_index("c")
    sid = lax.axis_index("s")
    wid = sid * NC + cid
    row0 = pl.ds(sid * RPT, RPT)
    pltpu.sync_copy(z_h.at[row0], acc.at[row0])
    pltpu.sync_copy(srcp_h.at[wid], src_v)
    pltpu.sync_copy(dstp_h.at[wid], dst_v)
    pltpu.sync_copy(ewp_h.at[wid], ew_v)
    plsc.subcore_barrier()

    @pl.loop(0, NCH)
    def _chunk(j):
        pltpu.sync_copy(feat_h.at[src_v.at[j]], rows_v)
        for g in range(CH // 16):
            ewg = ew_v[j, pl.ds(g * 16, 16)]
            for e2 in range(16):
                e = g * 16 + e2
                w = ewg[e2]
                for f in range(width // 16):
                    sl = pl.ds(f * 16, 16)
                    rows_v[e, sl] = rows_v[e, sl] * w
        pltpu.sync_copy(rows_v, acc.at[dst_v.at[j]], add=True)

    plsc.subcore_barrier()
    pltpu.sync_copy(acc.at[row0], outp_h.at[cid, row0])


@functools.lru_cache(maxsize=None)
def _edge_call(width):
    return pl.kernel(
        functools.partial(_edge_body, width),
        out_type=jax.ShapeDtypeStruct((NC, NP, width), jnp.float32),
        mesh=_sc_mesh(),
        compiler_params=pltpu.CompilerParams(use_tc_tiling_on_sc=False),
        scratch_types=[
            pltpu.VMEM((NCH, CH), jnp.int32),
            pltpu.VMEM((NCH, CH), jnp.int32),
            pltpu.VMEM((NCH, CH), jnp.float32),
            pltpu.VMEM((CH, width), jnp.float32),
            pltpu.VMEM_SHARED((NP, width), jnp.float32),
        ],
    )


_BM = 1000


def _tc1_body(dp_ref, x_ref, w1_ref, dinv_ref, hs_ref):
    degc = dp_ref[0] + dp_ref[1] + 1.0
    dinv1 = lax.rsqrt(degc)[:, 0:1]
    h = jnp.dot(x_ref[...], w1_ref[...], preferred_element_type=jnp.float32)
    dinv_ref[...] = dinv1
    hs_ref[...] = h * dinv1


def _tc2_body(t1p_ref, hs_ref, dinv_ref, b1_ref, w2p_ref, emb_ref, gs_ref):
    dinv1 = dinv_ref[...]
    s = t1p_ref[0] + t1p_ref[1] + hs_ref[...]
    emb = jnp.maximum(dinv1 * s + b1_ref[...], 0.0)
    emb_ref[...] = emb
    g = jnp.dot(emb, w2p_ref[...], preferred_element_type=jnp.float32)
    gs_ref[...] = g * dinv1


def _tc3_body(t2p_ref, gs_ref, dinv_ref, b2p_ref, out_ref):
    s = t2p_ref[0] + t2p_ref[1] + gs_ref[...]
    out16 = dinv_ref[...] * s + b2p_ref[...]
    out_ref[...] = out16[:, 0:2]


def _pspec(block, imap):
    return pl.BlockSpec(block, imap)


_tc1 = pl.pallas_call(
    _tc1_body,
    grid=(N // _BM,),
    in_specs=[
        _pspec((NC, _BM, G), lambda i: (0, i, 0)),
        _pspec((_BM, D), lambda i: (i, 0)),
        _pspec((D, H1), lambda i: (0, 0)),
    ],
    out_specs=[
        _pspec((_BM, 1), lambda i: (i, 0)),
        _pspec((_BM, H1), lambda i: (i, 0)),
    ],
    out_shape=[
        jax.ShapeDtypeStruct((N, 1), jnp.float32),
        jax.ShapeDtypeStruct((N, H1), jnp.float32),
    ],
)

_tc2 = pl.pallas_call(
    _tc2_body,
    grid=(N // _BM,),
    in_specs=[
        _pspec((NC, _BM, H1), lambda i: (0, i, 0)),
        _pspec((_BM, H1), lambda i: (i, 0)),
        _pspec((_BM, 1), lambda i: (i, 0)),
        _pspec((1, H1), lambda i: (0, 0)),
        _pspec((H1, G), lambda i: (0, 0)),
    ],
    out_specs=[
        _pspec((_BM, H1), lambda i: (i, 0)),
        _pspec((_BM, G), lambda i: (i, 0)),
    ],
    out_shape=[
        jax.ShapeDtypeStruct((N, H1), jnp.float32),
        jax.ShapeDtypeStruct((N, G), jnp.float32),
    ],
)

_tc3 = pl.pallas_call(
    _tc3_body,
    grid=(N // _BM,),
    in_specs=[
        _pspec((NC, _BM, G), lambda i: (0, i, 0)),
        _pspec((_BM, G), lambda i: (i, 0)),
        _pspec((_BM, 1), lambda i: (i, 0)),
        _pspec((1, G), lambda i: (0, 0)),
    ],
    out_specs=_pspec((_BM, 2), lambda i: (i, 0)),
    out_shape=jax.ShapeDtypeStruct((N, 2), jnp.float32),
)


def kernel(x, edge_index, edge_weight, W1, b1, W2, b2):
    pad = EP - edge_weight.shape[0]
    srcp = jnp.concatenate([edge_index[0], jnp.zeros((pad,), jnp.int32)])
    dstp = jnp.concatenate([edge_index[1], jnp.zeros((pad,), jnp.int32)])
    ewp = jnp.concatenate([edge_weight, jnp.zeros((pad,), jnp.float32)])
    srcp = srcp.reshape(NW, NCH, CH)
    dstp = dstp.reshape(NW, NCH, CH)
    ewp = ewp.reshape(NW, NCH, CH)
    z16 = jnp.zeros((NP, G), jnp.float32)
    z32 = jnp.zeros((NP, H1), jnp.float32)
    b1r = b1.reshape(1, H1)
    w2p = jnp.pad(W2, ((0, 0), (0, G - W2.shape[1])))
    b2p = jnp.pad(b2.reshape(1, -1), ((0, 0), (0, G - b2.shape[0])))

    degp = _deg_call()(dstp, ewp, z16)
    dinv, hs = _tc1(degp, x, W1)
    t1p = _edge_call(H1)(srcp, dstp, ewp, hs, z32)
    emb, gs = _tc2(t1p, hs, dinv, b1r, w2p)
    t2p = _edge_call(G)(srcp, dstp, ewp, gs, z16)
    out = _tc3(t2p, gs, dinv, b2p)
    return (emb, out)


# speedup vs baseline: 19.8976x; 1.2208x over previous
import functools

import jax
import jax.numpy as jnp
from jax import lax
from jax.experimental import pallas as pl
from jax.experimental.pallas import tpu as pltpu
from jax.experimental.pallas import tpu_sc as plsc

N = 10000
D = 256
H1 = 32
G = 16
NC = 2
NS = 16
NW = NC * NS
CH = 128
NCH = 40
EPW = NCH * CH
EP = NW * EPW
NP = 10240
RPT = NP // NS

@functools.lru_cache(maxsize=None)
def _sc_mesh():
    return plsc.VectorSubcoreMesh(
        core_axis_name="c", subcore_axis_name="s", num_cores=NC, num_subcores=NS
    )


def _deg_body(dstp_h, ewp_h, z16_h, degp_h, dst_v, ew_v, msg_v, acc):
    cid = lax.axis_index("c")
    sid = lax.axis_index("s")
    wid = sid * NC + cid
    row0 = pl.ds(sid * RPT, RPT)
    pltpu.sync_copy(z16_h.at[row0], acc.at[row0])
    pltpu.sync_copy(dstp_h.at[wid], dst_v)
    pltpu.sync_copy(ewp_h.at[wid], ew_v)
    plsc.subcore_barrier()

    @pl.loop(0, NCH)
    def _chunk(j):
        for g in range(CH // 16):
            ewg = ew_v[j, pl.ds(g * 16, 16)]
            for e2 in range(16):
                w = ewg[e2]
                msg_v[g * 16 + e2, :] = lax.broadcast_in_dim(w, (16,), ())
        pltpu.sync_copy(msg_v, acc.at[dst_v.at[j]], add=True)

    plsc.subcore_barrier()
    pltpu.sync_copy(acc.at[row0], degp_h.at[cid, row0])


@functools.lru_cache(maxsize=None)
def _deg_call():
    return pl.kernel(
        _deg_body,
        out_type=jax.ShapeDtypeStruct((NC, NP, G), jnp.float32),
        mesh=_sc_mesh(),
        compiler_params=pltpu.CompilerParams(use_tc_tiling_on_sc=False),
        scratch_types=[
            pltpu.VMEM((NCH, CH), jnp.int32),
            pltpu.VMEM((NCH, CH), jnp.float32),
            pltpu.VMEM((CH, G), jnp.float32),
            pltpu.VMEM_SHARED((NP, G), jnp.float32),
        ],
    )


def _edge_body(width, srcp_h, dstp_h, ewp_h, feat_h, z_h, outp_h,
               src_v, dst_v, ew_v, rows0_v, rows1_v, sem0, sem1, acc):
    cid = lax.axis_index("c")
    sid = lax.axis_index("s")
    wid = sid * NC + cid
    row0 = pl.ds(sid * RPT, RPT)
    pltpu.sync_copy(z_h.at[row0], acc.at[row0])
    pltpu.sync_copy(srcp_h.at[wid], src_v)
    pltpu.sync_copy(dstp_h.at[wid], dst_v)
    pltpu.sync_copy(ewp_h.at[wid], ew_v)
    plsc.subcore_barrier()

    bufs = (rows0_v, rows1_v)
    sems = (sem0, sem1)

    def _gather(j, b):
        pltpu.async_copy(feat_h.at[src_v.at[j]], bufs[b], sems[b])

    def _consume(j, b):
        rows_v = bufs[b]
        pltpu.make_async_copy(feat_h.at[src_v.at[j]], rows_v, sems[b]).wait()
        for g in range(CH // 16):
            ewg = ew_v[j, pl.ds(g * 16, 16)]
            for e2 in range(16):
                e = g * 16 + e2
                w = ewg[e2]
                for f in range(width // 16):
                    sl = pl.ds(f * 16, 16)
                    rows_v[e, sl] = rows_v[e, sl] * w
        pltpu.sync_copy(rows_v, acc.at[dst_v.at[j]], add=True)

    _gather(0, 0)
    _gather(1, 1)

    @pl.loop(0, NCH - 2, step=2)
    def _chunk(j):
        _consume(j, 0)
        _gather(j + 2, 0)
        _consume(j + 1, 1)
        _gather(j + 3, 1)

    _consume(NCH - 2, 0)
    _consume(NCH - 1, 1)

    plsc.subcore_barrier()
    pltpu.sync_copy(acc.at[row0], outp_h.at[cid, row0])


@functools.lru_cache(maxsize=None)
def _edge_call(width):
    return pl.kernel(
        functools.partial(_edge_body, width),
        out_type=jax.ShapeDtypeStruct((NC, NP, width), jnp.float32),
        mesh=_sc_mesh(),
        compiler_params=pltpu.CompilerParams(use_tc_tiling_on_sc=False),
        scratch_types=[
            pltpu.VMEM((NCH, CH), jnp.int32),
            pltpu.VMEM((NCH, CH), jnp.int32),
            pltpu.VMEM((NCH, CH), jnp.float32),
            pltpu.VMEM((CH, width), jnp.float32),
            pltpu.VMEM((CH, width), jnp.float32),
            pltpu.SemaphoreType.DMA,
            pltpu.SemaphoreType.DMA,
            pltpu.VMEM_SHARED((NP, width), jnp.float32),
        ],
    )


_BM = 1000


def _tc1_body(dp_ref, x_ref, w1_ref, dinv_ref, hs_ref):
    degc = dp_ref[0] + dp_ref[1] + 1.0
    dinv1 = lax.rsqrt(degc)[:, 0:1]
    h = jnp.dot(x_ref[...], w1_ref[...], preferred_element_type=jnp.float32)
    dinv_ref[...] = dinv1
    hs_ref[...] = h * dinv1


def _tc2_body(t1p_ref, hs_ref, dinv_ref, b1_ref, w2p_ref, emb_ref, gs_ref):
    dinv1 = dinv_ref[...]
    s = t1p_ref[0] + t1p_ref[1] + hs_ref[...]
    emb = jnp.maximum(dinv1 * s + b1_ref[...], 0.0)
    emb_ref[...] = emb
    g = jnp.dot(emb, w2p_ref[...], preferred_element_type=jnp.float32)
    gs_ref[...] = g * dinv1


def _tc3_body(t2p_ref, gs_ref, dinv_ref, b2p_ref, out_ref):
    s = t2p_ref[0] + t2p_ref[1] + gs_ref[...]
    out16 = dinv_ref[...] * s + b2p_ref[...]
    out_ref[...] = out16[:, 0:2]


def _pspec(block, imap):
    return pl.BlockSpec(block, imap)


_tc1 = pl.pallas_call(
    _tc1_body,
    grid=(N // _BM,),
    in_specs=[
        _pspec((NC, _BM, G), lambda i: (0, i, 0)),
        _pspec((_BM, D), lambda i: (i, 0)),
        _pspec((D, H1), lambda i: (0, 0)),
    ],
    out_specs=[
        _pspec((_BM, 1), lambda i: (i, 0)),
        _pspec((_BM, H1), lambda i: (i, 0)),
    ],
    out_shape=[
        jax.ShapeDtypeStruct((N, 1), jnp.float32),
        jax.ShapeDtypeStruct((N, H1), jnp.float32),
    ],
)

_tc2 = pl.pallas_call(
    _tc2_body,
    grid=(N // _BM,),
    in_specs=[
        _pspec((NC, _BM, H1), lambda i: (0, i, 0)),
        _pspec((_BM, H1), lambda i: (i, 0)),
        _pspec((_BM, 1), lambda i: (i, 0)),
        _pspec((1, H1), lambda i: (0, 0)),
        _pspec((H1, G), lambda i: (0, 0)),
    ],
    out_specs=[
        _pspec((_BM, H1), lambda i: (i, 0)),
        _pspec((_BM, G), lambda i: (i, 0)),
    ],
    out_shape=[
        jax.ShapeDtypeStruct((N, H1), jnp.float32),
        jax.ShapeDtypeStruct((N, G), jnp.float32),
    ],
)

_tc3 = pl.pallas_call(
    _tc3_body,
    grid=(N // _BM,),
    in_specs=[
        _pspec((NC, _BM, G), lambda i: (0, i, 0)),
        _pspec((_BM, G), lambda i: (i, 0)),
        _pspec((_BM, 1), lambda i: (i, 0)),
        _pspec((1, G), lambda i: (0, 0)),
    ],
    out_specs=_pspec((_BM, 2), lambda i: (i, 0)),
    out_shape=jax.ShapeDtypeStruct((N, 2), jnp.float32),
)


def kernel(x, edge_index, edge_weight, W1, b1, W2, b2):
    pad = EP - edge_weight.shape[0]
    srcp = jnp.concatenate([edge_index[0], jnp.zeros((pad,), jnp.int32)])
    dstp = jnp.concatenate([edge_index[1], jnp.zeros((pad,), jnp.int32)])
    ewp = jnp.concatenate([edge_weight, jnp.zeros((pad,), jnp.float32)])
    srcp = srcp.reshape(NW, NCH, CH)
    dstp = dstp.reshape(NW, NCH, CH)
    ewp = ewp.reshape(NW, NCH, CH)
    z16 = jnp.zeros((NP, G), jnp.float32)
    z32 = jnp.zeros((NP, H1), jnp.float32)
    b1r = b1.reshape(1, H1)
    w2p = jnp.pad(W2, ((0, 0), (0, G - W2.shape[1])))
    b2p = jnp.pad(b2.reshape(1, -1), ((0, 0), (0, G - b2.shape[0])))

    degp = _deg_call()(dstp, ewp, z16)
    dinv, hs = _tc1(degp, x, W1)
    t1p = _edge_call(H1)(srcp, dstp, ewp, hs, z32)
    emb, gs = _tc2(t1p, hs, dinv, b1r, w2p)
    t2p = _edge_call(G)(srcp, dstp, ewp, gs, z16)
    out = _tc3(t2p, gs, dinv, b2p)
    return (emb, out)


# speedup vs baseline: 25.4247x; 1.2778x over previous
import functools

import jax
import jax.numpy as jnp
from jax import lax
from jax.experimental import pallas as pl
from jax.experimental.pallas import tpu as pltpu
from jax.experimental.pallas import tpu_sc as plsc

N = 10000
E = 160000
D = 256
H1 = 32
G = 16
NC = 2
NS = 16
NW = NC * NS
CH = 128
NCHT = 39
EPT = NCHT * CH
XBASE = NW * EPT
NP = 10240
RPT = NP // NS


@functools.lru_cache(maxsize=None)
def _sc_mesh():
    return plsc.VectorSubcoreMesh(
        core_axis_name="c", subcore_axis_name="s", num_cores=NC, num_subcores=NS
    )


def _scale_rows(rows_v, ew_v, off, width, nedge=CH):
    for g in range(nedge // 16):
        ewg = ew_v[pl.ds(off + g * 16, 16)]
        for e2 in range(16):
            e = g * 16 + e2
            w = ewg[e2]
            for f in range(width // 16):
                sl = pl.ds(f * 16, 16)
                rows_v[e, sl] = rows_v[e, sl] * w


def _splat_rows(msg_v, ew_v, off, nedge=CH):
    for g in range(nedge // 16):
        ewg = ew_v[pl.ds(off + g * 16, 16)]
        for e2 in range(16):
            w = ewg[e2]
            msg_v[g * 16 + e2, :] = lax.broadcast_in_dim(w, (16,), ())


def _deg_body(ei_h, ew_h, z16_h, degp_h, dst_v, ew_v, xdst_v, xew_v, msg_v, acc):
    cid = lax.axis_index("c")
    sid = lax.axis_index("s")
    wid = sid * NC + cid
    row0 = pl.ds(sid * RPT, RPT)
    base = wid * EPT
    pltpu.sync_copy(z16_h.at[row0], acc.at[row0])
    pltpu.sync_copy(ei_h.at[1, pl.ds(base, EPT)], dst_v)
    pltpu.sync_copy(ew_h.at[pl.ds(base, EPT)], ew_v)

    @pl.when(wid < NC)
    def _():
        pltpu.sync_copy(ei_h.at[1, pl.ds(XBASE + wid * CH, CH)], xdst_v)
        pltpu.sync_copy(ew_h.at[pl.ds(XBASE + wid * CH, CH)], xew_v)

    plsc.subcore_barrier()

    @pl.loop(0, NCHT)
    def _chunk(j):
        _splat_rows(msg_v, ew_v, j * CH)
        pltpu.sync_copy(msg_v, acc.at[dst_v.at[pl.ds(j * CH, CH)]], add=True)

    @pl.when(wid < NC)
    def _():
        _splat_rows(msg_v, xew_v, 0)
        pltpu.sync_copy(msg_v, acc.at[xdst_v], add=True)

    plsc.subcore_barrier()
    pltpu.sync_copy(acc.at[row0], degp_h.at[cid, row0])


@functools.lru_cache(maxsize=None)
def _deg_call():
    return pl.kernel(
        _deg_body,
        out_type=jax.ShapeDtypeStruct((NC, NP, G), jnp.float32),
        mesh=_sc_mesh(),
        compiler_params=pltpu.CompilerParams(
            use_tc_tiling_on_sc=False, disable_bounds_checks=True),
        scratch_types=[
            pltpu.VMEM((EPT,), jnp.int32),
            pltpu.VMEM((EPT,), jnp.float32),
            pltpu.VMEM((CH,), jnp.int32),
            pltpu.VMEM((CH,), jnp.float32),
            pltpu.VMEM((CH, G), jnp.float32),
            pltpu.VMEM_SHARED((NP, G), jnp.float32),
        ],
    )


def _edge_body(width, ei_h, ew_h, feat_h, z_h, outp_h,
               src_v, dst_v, ew_v, xsrc_v, xdst_v, xew_v,
               rows0_v, rows1_v, rowsx_v, sem0, sem1, semx, acc):
    cid = lax.axis_index("c")
    sid = lax.axis_index("s")
    wid = sid * NC + cid
    row0 = pl.ds(sid * RPT, RPT)
    base = wid * EPT
    pltpu.sync_copy(z_h.at[row0], acc.at[row0])
    pltpu.sync_copy(ei_h.at[0, pl.ds(base, EPT)], src_v)
    pltpu.sync_copy(ei_h.at[1, pl.ds(base, EPT)], dst_v)
    pltpu.sync_copy(ew_h.at[pl.ds(base, EPT)], ew_v)

    @pl.when(wid < NC)
    def _():
        pltpu.sync_copy(ei_h.at[0, pl.ds(XBASE + wid * CH, CH)], xsrc_v)
        pltpu.sync_copy(ei_h.at[1, pl.ds(XBASE + wid * CH, CH)], xdst_v)
        pltpu.sync_copy(ew_h.at[pl.ds(XBASE + wid * CH, CH)], xew_v)

    plsc.subcore_barrier()

    bufs = (rows0_v, rows1_v)
    sems = (sem0, sem1)

    def _gather(j, b):
        pltpu.async_copy(feat_h.at[src_v.at[pl.ds(j * CH, CH)]], bufs[b], sems[b])

    def _consume(j, b):
        rows_v = bufs[b]
        pltpu.make_async_copy(
            feat_h.at[src_v.at[pl.ds(j * CH, CH)]], rows_v, sems[b]).wait()
        _scale_rows(rows_v, ew_v, j * CH, width)
        pltpu.sync_copy(rows_v, acc.at[dst_v.at[pl.ds(j * CH, CH)]], add=True)

    _gather(0, 0)
    _gather(1, 1)

    @pl.when(wid < NC)
    def _():
        pltpu.async_copy(feat_h.at[xsrc_v], rowsx_v, semx)

    @pl.loop(0, NCHT - 3, step=2)
    def _chunk(j):
        _consume(j, 0)
        _gather(j + 2, 0)
        _consume(j + 1, 1)
        _gather(j + 3, 1)

    _consume(NCHT - 3, 0)
    _gather(NCHT - 1, 0)
    _consume(NCHT - 2, 1)
    _consume(NCHT - 1, 0)

    @pl.when(wid < NC)
    def _():
        pltpu.make_async_copy(feat_h.at[xsrc_v], rowsx_v, semx).wait()
        _scale_rows(rowsx_v, xew_v, 0, width)
        pltpu.sync_copy(rowsx_v, acc.at[xdst_v], add=True)

    plsc.subcore_barrier()
    pltpu.sync_copy(acc.at[row0], outp_h.at[cid, row0])


@functools.lru_cache(maxsize=None)
def _edge_call(width):
    return pl.kernel(
        functools.partial(_edge_body, width),
        out_type=jax.ShapeDtypeStruct((NC, NP, width), jnp.float32),
        mesh=_sc_mesh(),
        compiler_params=pltpu.CompilerParams(
            use_tc_tiling_on_sc=False, disable_bounds_checks=True),
        scratch_types=[
            pltpu.VMEM((EPT,), jnp.int32),
            pltpu.VMEM((EPT,), jnp.int32),
            pltpu.VMEM((EPT,), jnp.float32),
            pltpu.VMEM((CH,), jnp.int32),
            pltpu.VMEM((CH,), jnp.int32),
            pltpu.VMEM((CH,), jnp.float32),
            pltpu.VMEM((CH, width), jnp.float32),
            pltpu.VMEM((CH, width), jnp.float32),
            pltpu.VMEM((CH, width), jnp.float32),
            pltpu.SemaphoreType.DMA,
            pltpu.SemaphoreType.DMA,
            pltpu.SemaphoreType.DMA,
            pltpu.VMEM_SHARED((NP, width), jnp.float32),
        ],
    )


_BM = 1000


def _tc0_body(x_ref, w1_ref, h_ref):
    h_ref[...] = jnp.dot(x_ref[...], w1_ref[...],
                         preferred_element_type=jnp.float32)


def _tc1_body(dp_ref, h_ref, dinv_ref, hs_ref):
    degc = dp_ref[0] + dp_ref[1] + 1.0
    dinv1 = lax.rsqrt(degc)[:, 0:1]
    dinv_ref[...] = dinv1
    hs_ref[...] = h_ref[...] * dinv1


def _tc2_body(t1p_ref, hs_ref, dinv_ref, b1_ref, w2p_ref, emb_ref, gs_ref):
    dinv1 = dinv_ref[...]
    s = t1p_ref[0] + t1p_ref[1] + hs_ref[...]
    emb = jnp.maximum(dinv1 * s + b1_ref[...], 0.0)
    emb_ref[...] = emb
    g = jnp.dot(emb, w2p_ref[...], preferred_element_type=jnp.float32)
    gs_ref[...] = g * dinv1


def _tc3_body(t2p_ref, gs_ref, dinv_ref, b2p_ref, out_ref):
    s = t2p_ref[0] + t2p_ref[1] + gs_ref[...]
    out16 = dinv_ref[...] * s + b2p_ref[...]
    out_ref[...] = out16[:, 0:2]


def _pspec(block, imap):
    return pl.BlockSpec(block, imap)


_tc0 = pl.pallas_call(
    _tc0_body,
    grid=(N // _BM,),
    in_specs=[
        _pspec((_BM, D), lambda i: (i, 0)),
        _pspec((D, H1), lambda i: (0, 0)),
    ],
    out_specs=_pspec((_BM, H1), lambda i: (i, 0)),
    out_shape=jax.ShapeDtypeStruct((N, H1), jnp.float32),
)

_tc1 = pl.pallas_call(
    _tc1_body,
    grid=(N // _BM,),
    in_specs=[
        _pspec((NC, _BM, G), lambda i: (0, i, 0)),
        _pspec((_BM, H1), lambda i: (i, 0)),
    ],
    out_specs=[
        _pspec((_BM, 1), lambda i: (i, 0)),
        _pspec((_BM, H1), lambda i: (i, 0)),
    ],
    out_shape=[
        jax.ShapeDtypeStruct((N, 1), jnp.float32),
        jax.ShapeDtypeStruct((N, H1), jnp.float32),
    ],
)

_tc2 = pl.pallas_call(
    _tc2_body,
    grid=(N // _BM,),
    in_specs=[
        _pspec((NC, _BM, H1), lambda i: (0, i, 0)),
        _pspec((_BM, H1), lambda i: (i, 0)),
        _pspec((_BM, 1), lambda i: (i, 0)),
        _pspec((1, H1), lambda i: (0, 0)),
        _pspec((H1, G), lambda i: (0, 0)),
    ],
    out_specs=[
        _pspec((_BM, H1), lambda i: (i, 0)),
        _pspec((_BM, G), lambda i: (i, 0)),
    ],
    out_shape=[
        jax.ShapeDtypeStruct((N, H1), jnp.float32),
        jax.ShapeDtypeStruct((N, G), jnp.float32),
    ],
)

_tc3 = pl.pallas_call(
    _tc3_body,
    grid=(N // _BM,),
    in_specs=[
        _pspec((NC, _BM, G), lambda i: (0, i, 0)),
        _pspec((_BM, G), lambda i: (i, 0)),
        _pspec((_BM, 1), lambda i: (i, 0)),
        _pspec((1, G), lambda i: (0, 0)),
    ],
    out_specs=_pspec((_BM, 2), lambda i: (i, 0)),
    out_shape=jax.ShapeDtypeStruct((N, 2), jnp.float32),
)


def kernel(x, edge_index, edge_weight, W1, b1, W2, b2):
    z16 = jnp.zeros((NP, G), jnp.float32)
    z32 = jnp.zeros((NP, H1), jnp.float32)
    b1r = b1.reshape(1, H1)
    w2p = jnp.pad(W2, ((0, 0), (0, G - W2.shape[1])))
    b2p = jnp.pad(b2.reshape(1, -1), ((0, 0), (0, G - b2.shape[0])))

    h = _tc0(x, W1)
    degp = _deg_call()(edge_index, edge_weight, z16)
    dinv, hs = _tc1(degp, h)
    t1p = _edge_call(H1)(edge_index, edge_weight, hs, z32)
    emb, gs = _tc2(t1p, hs, dinv, b1r, w2p)
    t2p = _edge_call(G)(edge_index, edge_weight, gs, z16)
    out = _tc3(t2p, gs, dinv, b2p)
    return (emb, out)
